# Initial kernel scaffold; baseline (speedup 1.0000x reference)
import jax
import jax.numpy as jnp
from jax.experimental import pallas as pl


def kernel(x, pos, batch, params):
    raise NotImplementedError("write your pallas kernel here")


# speedup vs baseline: 2.1356x; 2.1356x over previous
import functools
import jax
import jax.numpy as jnp
from jax.experimental import pallas as pl
from jax.experimental.pallas import tpu as pltpu

_EPS = 1e-5
_MAX_NB = 64


def _fps_body(n_pts, n_samp, px_ref, py_ref, pz_ref, sel_ref, qx_ref, qy_ref,
              qz_ref):
    px = px_ref[...]
    py = py_ref[...]
    pz = pz_ref[...]
    nb = px.shape[0]
    lane = jax.lax.broadcasted_iota(jnp.int32, (nb, n_pts), 1)
    lane_q = jax.lax.broadcasted_iota(jnp.int32, (nb, n_samp), 1)

    def step(i, carry):
        last, dist, sel, qx, qy, qz = carry
        is_last = lane == last
        lx = jnp.sum(jnp.where(is_last, px, 0.0), axis=1, keepdims=True)
        ly = jnp.sum(jnp.where(is_last, py, 0.0), axis=1, keepdims=True)
        lz = jnp.sum(jnp.where(is_last, pz, 0.0), axis=1, keepdims=True)
        rec = lane_q == i
        qx = jnp.where(rec, lx, qx)
        qy = jnp.where(rec, ly, qy)
        qz = jnp.where(rec, lz, qz)
        d = (px - lx) ** 2 + (py - ly) ** 2 + (pz - lz) ** 2
        dist = jnp.minimum(dist, d)
        m = jnp.max(dist, axis=1, keepdims=True)
        nxt = jnp.min(jnp.where(dist >= m, lane, 2 ** 30), axis=1,
                      keepdims=True)
        sel = jnp.where(lane_q == i + 1, nxt, sel)
        return nxt, dist, sel, qx, qy, qz

    init = (jnp.zeros((nb, 1), jnp.int32),
            jnp.full((nb, n_pts), jnp.inf, jnp.float32),
            jnp.zeros((nb, n_samp), jnp.int32),
            jnp.zeros((nb, n_samp), jnp.float32),
            jnp.zeros((nb, n_samp), jnp.float32),
            jnp.zeros((nb, n_samp), jnp.float32))
    _, _, sel, qx, qy, qz = jax.lax.fori_loop(0, n_samp, step, init)
    sel_ref[...] = sel
    qx_ref[...] = qx
    qy_ref[...] = qy
    qz_ref[...] = qz


def _fps(px, py, pz, n_samp):
    nb, n_pts = px.shape
    out = (jax.ShapeDtypeStruct((nb, n_samp), jnp.int32),
           jax.ShapeDtypeStruct((nb, n_samp), jnp.float32),
           jax.ShapeDtypeStruct((nb, n_samp), jnp.float32),
           jax.ShapeDtypeStruct((nb, n_samp), jnp.float32))
    return pl.pallas_call(
        functools.partial(_fps_body, n_pts, n_samp),
        out_shape=out,
    )(px, py, pz)


def _u_body(x_ref, p_ref, wx_ref, wp_ref, o_ref):
    x = x_ref[...]
    p = p_ref[...]
    u = jnp.dot(x, wx_ref[...], preferred_element_type=jnp.float32)
    u += jnp.dot(p, wp_ref[...], preferred_element_type=jnp.float32)
    o_ref[...] = u


def _compute_u(x2d, p2d, wx, wp):
    n = x2d.shape[0]
    d = wx.shape[1]
    return pl.pallas_call(
        _u_body,
        out_shape=jax.ShapeDtypeStruct((n, d), jnp.float32),
    )(x2d, p2d, wx, wp)


def _max_mid(m3):
    k = m3.shape[1]
    while k > 1:
        h = k // 2
        m3 = jnp.maximum(m3[:, :h, :], m3[:, h:, :])
        k = h
    return m3[:, 0, :]


def _pair_body(qb, g_ref, posq_ref, maskf_ref, wp_ref, b1_ref, w2_ref,
               b2_ref, w3_ref, b3_ref, o_ref):
    k = _MAX_NB
    d1 = g_ref.shape[2]
    posq = posq_ref[0]
    c = b1_ref[...] - jnp.dot(posq, wp_ref[...],
                              preferred_element_type=jnp.float32)
    g = g_ref[0].reshape(qb, k, d1)
    a1 = jnp.maximum(g + c[:, None, :], 0.0)
    a1 = a1.reshape(qb * k, d1)
    a2 = jnp.dot(a1, w2_ref[...], preferred_element_type=jnp.float32)
    a2 = jnp.maximum(a2 + b2_ref[...], 0.0)
    m = jnp.dot(a2, w3_ref[...], preferred_element_type=jnp.float32)
    m = m + b3_ref[...]
    dout = m.shape[1]
    m = m.reshape(qb, k, dout) + maskf_ref[0][:, :, None]
    o_ref[0] = _max_mid(m)


def _pair_mlp(g, posq, maskf, wp, b1, w2, b2, w3, b3, qb):
    nb, q, _ = posq.shape
    k = _MAX_NB
    d1 = g.shape[2]
    dh = w2.shape[1]
    dout = w3.shape[1]
    grid = (nb, q // qb)
    return pl.pallas_call(
        functools.partial(_pair_body, qb),
        grid=grid,
        in_specs=[
            pl.BlockSpec((1, qb * k, d1), lambda b, t: (b, t, 0)),
            pl.BlockSpec((1, qb, 3), lambda b, t: (b, t, 0)),
            pl.BlockSpec((1, qb, k), lambda b, t: (b, t, 0)),
            pl.BlockSpec((3, d1), lambda b, t: (0, 0)),
            pl.BlockSpec((1, d1), lambda b, t: (0, 0)),
            pl.BlockSpec((d1, dh), lambda b, t: (0, 0)),
            pl.BlockSpec((1, dh), lambda b, t: (0, 0)),
            pl.BlockSpec((dh, dout), lambda b, t: (0, 0)),
            pl.BlockSpec((1, dout), lambda b, t: (0, 0)),
        ],
        out_specs=pl.BlockSpec((1, qb, dout), lambda b, t: (b, t, 0)),
        out_shape=jax.ShapeDtypeStruct((nb, q, dout), jnp.float32),
    )(g, posq, maskf, wp, b1, w2, b2, w3, b3)


def _sa3_body(n_cloud, q, x_ref, p_ref, w0x_ref, w0p_ref, b0_ref, w1_ref,
              b1_ref, w2_ref, b2_ref, h0_ref, hb0_ref, h1_ref, hb1_ref,
              h2_ref, hb2_ref, o_ref):
    x = x_ref[...]
    p = p_ref[...]
    h = jnp.dot(x, w0x_ref[...], preferred_element_type=jnp.float32)
    h += jnp.dot(p, w0p_ref[...], preferred_element_type=jnp.float32)
    h = jnp.maximum(h + b0_ref[...], 0.0)
    h = jnp.dot(h, w1_ref[...], preferred_element_type=jnp.float32)
    h = jnp.maximum(h + b1_ref[...], 0.0)
    h = jnp.dot(h, w2_ref[...], preferred_element_type=jnp.float32)
    h = h + b2_ref[...]
    h = h.reshape(n_cloud, q, h.shape[1])
    g = _max_mid(h)
    g = jnp.dot(g, h0_ref[...], preferred_element_type=jnp.float32)
    g = jnp.maximum(g + hb0_ref[...], 0.0)
    g = jnp.dot(g, h1_ref[...], preferred_element_type=jnp.float32)
    g = jnp.maximum(g + hb1_ref[...], 0.0)
    g = jnp.dot(g, h2_ref[...], preferred_element_type=jnp.float32)
    g = g + hb2_ref[...]
    mx = jnp.max(g, axis=1, keepdims=True)
    lse = jnp.log(jnp.sum(jnp.exp(g - mx), axis=1, keepdims=True)) + mx
    o_ref[...] = g - lse


def _sa3_head(x2d, p2d, n_cloud, q, args):
    ncls = args[-1].shape[1]
    return pl.pallas_call(
        functools.partial(_sa3_body, n_cloud, q),
        out_shape=jax.ShapeDtypeStruct((n_cloud, ncls), jnp.float32),
    )(x2d, p2d, *args)


def _fold_norm(params):
    s = (1.0 + _EPS) ** 0.5
    ws, bs = [], []
    n = len(params["W"])
    for i in range(n):
        w, b = params["W"][i], params["b"][i]
        if i < n - 1:
            g, beta = params["g"][i], params["beta"][i]
            ws.append(w * (g / s)[None, :])
            bs.append(b * g / s + beta)
        else:
            ws.append(w)
            bs.append(b)
    return ws, bs


def _select_neighbors(posq, px, py, pz, r):
    d2 = ((posq[:, :, 0:1] - px[:, None, :]) ** 2
          + (posq[:, :, 1:2] - py[:, None, :]) ** 2
          + (posq[:, :, 2:3] - pz[:, None, :]) ** 2)
    score = jnp.where(d2 <= r * r, -d2, -jnp.inf)
    vals, idx = jax.lax.top_k(score, _MAX_NB)
    maskf = jnp.where(vals > -jnp.inf, 0.0, -jnp.inf).astype(jnp.float32)
    return idx.astype(jnp.int32), maskf


def _gather_pairs(u, idx):
    nb, q, k = idx.shape
    d = u.shape[2]
    flat = idx.reshape(nb, q * k)
    return jnp.take_along_axis(u, flat[:, :, None], axis=1)


def _sa_stage(xb, posb, px, py, pz, params, ratio, r, qb):
    nb, n_pts, fdim = xb.shape
    n_samp = int(n_pts * ratio)
    ws, bs = _fold_norm(params)
    w1, w2, w3 = ws
    b1, b2, b3 = bs
    d1 = w1.shape[1]

    sel, qx, qy, qz = _fps(px, py, pz, n_samp)
    posq = jnp.stack([qx, qy, qz], axis=-1)

    u = _compute_u(xb.reshape(nb * n_pts, fdim),
                   posb.reshape(nb * n_pts, 3),
                   w1[:fdim], w1[fdim:]).reshape(nb, n_pts, d1)

    idx, maskf = _select_neighbors(posq, px, py, pz, r)
    g = _gather_pairs(u, idx)

    x_out = _pair_mlp(g, posq, maskf, w1[fdim:], b1[None, :], w2,
                      b2[None, :], w3, b3[None, :], qb)
    return x_out, posq, qx, qy, qz


def kernel(x, pos, batch, params):
    nb = batch.shape[0] // 1024
    n_pts = 1024
    xb = x.reshape(nb, n_pts, -1)
    pb = pos.reshape(nb, n_pts, 3)
    px = pb[:, :, 0]
    py = pb[:, :, 1]
    pz = pb[:, :, 2]

    x1, posq1, q1x, q1y, q1z = _sa_stage(xb, pb, px, py, pz, params["sa1"],
                                         0.5, 0.2, 128)
    x2, posq2, _, _, _ = _sa_stage(x1, posq1, q1x, q1y, q1z, params["sa2"],
                                   0.25, 0.4, 128)

    ws, bs = _fold_norm(params["sa3"])
    hw, hb = _fold_norm(params["head"])
    q2 = x2.shape[1]
    f2 = x2.shape[2]
    args = (ws[0][:f2], ws[0][f2:], bs[0][None, :], ws[1], bs[1][None, :],
            ws[2], bs[2][None, :], hw[0], hb[0][None, :], hw[1],
            hb[1][None, :], hw[2], hb[2][None, :])
    return _sa3_head(x2.reshape(nb * q2, f2), posq2.reshape(nb * q2, 3),
                     nb, q2, args)


# speedup vs baseline: 14.2937x; 6.6929x over previous
import functools
import jax
import jax.numpy as jnp
from jax import lax
from jax.experimental import pallas as pl
from jax.experimental.pallas import tpu as pltpu
from jax.experimental.pallas import tpu_sc as plsc

_EPS = 1e-5
_MAX_NB = 64


def _fps_body(n_pts, n_samp, px_ref, py_ref, pz_ref, sel_ref, qx_ref, qy_ref,
              qz_ref):
    px = px_ref[...]
    py = py_ref[...]
    pz = pz_ref[...]
    nb = px.shape[0]
    lane = jax.lax.broadcasted_iota(jnp.int32, (nb, n_pts), 1)
    lane_q = jax.lax.broadcasted_iota(jnp.int32, (nb, n_samp), 1)

    def step(i, carry):
        last, dist, sel, qx, qy, qz = carry
        is_last = lane == last
        lx = jnp.sum(jnp.where(is_last, px, 0.0), axis=1, keepdims=True)
        ly = jnp.sum(jnp.where(is_last, py, 0.0), axis=1, keepdims=True)
        lz = jnp.sum(jnp.where(is_last, pz, 0.0), axis=1, keepdims=True)
        rec = lane_q == i
        qx = jnp.where(rec, lx, qx)
        qy = jnp.where(rec, ly, qy)
        qz = jnp.where(rec, lz, qz)
        d = (px - lx) ** 2 + (py - ly) ** 2 + (pz - lz) ** 2
        dist = jnp.minimum(dist, d)
        m = jnp.max(dist, axis=1, keepdims=True)
        nxt = jnp.min(jnp.where(dist >= m, lane, 2 ** 30), axis=1,
                      keepdims=True)
        sel = jnp.where(lane_q == i + 1, nxt, sel)
        return nxt, dist, sel, qx, qy, qz

    init = (jnp.zeros((nb, 1), jnp.int32),
            jnp.full((nb, n_pts), jnp.inf, jnp.float32),
            jnp.zeros((nb, n_samp), jnp.int32),
            jnp.zeros((nb, n_samp), jnp.float32),
            jnp.zeros((nb, n_samp), jnp.float32),
            jnp.zeros((nb, n_samp), jnp.float32))
    _, _, sel, qx, qy, qz = jax.lax.fori_loop(0, n_samp, step, init)
    sel_ref[...] = sel
    qx_ref[...] = qx
    qy_ref[...] = qy
    qz_ref[...] = qz


def _fps(px, py, pz, n_samp):
    nb, n_pts = px.shape
    out = (jax.ShapeDtypeStruct((nb, n_samp), jnp.int32),
           jax.ShapeDtypeStruct((nb, n_samp), jnp.float32),
           jax.ShapeDtypeStruct((nb, n_samp), jnp.float32),
           jax.ShapeDtypeStruct((nb, n_samp), jnp.float32))
    return pl.pallas_call(
        functools.partial(_fps_body, n_pts, n_samp),
        out_shape=out,
    )(px, py, pz)


def _u_body(x_ref, p_ref, wx_ref, wp_ref, o_ref):
    x = x_ref[...]
    p = p_ref[...]
    u = jnp.dot(x, wx_ref[...], preferred_element_type=jnp.float32)
    u += jnp.dot(p, wp_ref[...], preferred_element_type=jnp.float32)
    o_ref[...] = u


def _compute_u(x2d, p2d, wx, wp):
    n = x2d.shape[0]
    d = wx.shape[1]
    return pl.pallas_call(
        _u_body,
        out_shape=jax.ShapeDtypeStruct((n, d), jnp.float32),
    )(x2d, p2d, wx, wp)


def _max_mid(m3):
    k = m3.shape[1]
    while k > 1:
        h = k // 2
        m3 = jnp.maximum(m3[:, :h, :], m3[:, h:, :])
        k = h
    return m3[:, 0, :]


def _pair_body(qb, d1, g_ref, posq_ref, maskf_ref, wp_ref, b1_ref, w2_ref,
               b2_ref, w3_ref, b3_ref, o_ref):
    k = _MAX_NB
    dpad = g_ref.shape[2]
    posq = posq_ref[0]
    c = b1_ref[...] - jnp.dot(posq, wp_ref[...],
                              preferred_element_type=jnp.float32)
    g = g_ref[0].reshape(qb, k, dpad)
    if dpad != d1:
        g = g[:, :, :d1]
    a1 = jnp.maximum(g + c[:, None, :], 0.0)
    a1 = a1.reshape(qb * k, d1)
    a2 = jnp.dot(a1, w2_ref[...], preferred_element_type=jnp.float32)
    a2 = jnp.maximum(a2 + b2_ref[...], 0.0)
    m = jnp.dot(a2, w3_ref[...], preferred_element_type=jnp.float32)
    m = m + b3_ref[...]
    dout = m.shape[1]
    m = m.reshape(qb, k, dout) + maskf_ref[0][:, :, None]
    o_ref[0] = _max_mid(m)


def _pair_mlp(g, posq, maskf, wp, b1, w2, b2, w3, b3, qb):
    nb, q, _ = posq.shape
    k = _MAX_NB
    dpad = g.shape[2]
    d1 = w2.shape[0]
    dh = w2.shape[1]
    dout = w3.shape[1]
    grid = (nb, q // qb)
    return pl.pallas_call(
        functools.partial(_pair_body, qb, d1),
        grid=grid,
        in_specs=[
            pl.BlockSpec((1, qb * k, dpad), lambda b, t: (b, t, 0)),
            pl.BlockSpec((1, qb, 3), lambda b, t: (b, t, 0)),
            pl.BlockSpec((1, qb, k), lambda b, t: (b, t, 0)),
            pl.BlockSpec((3, d1), lambda b, t: (0, 0)),
            pl.BlockSpec((1, d1), lambda b, t: (0, 0)),
            pl.BlockSpec((d1, dh), lambda b, t: (0, 0)),
            pl.BlockSpec((1, dh), lambda b, t: (0, 0)),
            pl.BlockSpec((dh, dout), lambda b, t: (0, 0)),
            pl.BlockSpec((1, dout), lambda b, t: (0, 0)),
        ],
        out_specs=pl.BlockSpec((1, qb, dout), lambda b, t: (b, t, 0)),
        out_shape=jax.ShapeDtypeStruct((nb, q, dout), jnp.float32),
    )(g, posq, maskf, wp, b1, w2, b2, w3, b3)


def _sa3_body(n_cloud, q, x_ref, p_ref, w0x_ref, w0p_ref, b0_ref, w1_ref,
              b1_ref, w2_ref, b2_ref, h0_ref, hb0_ref, h1_ref, hb1_ref,
              h2_ref, hb2_ref, o_ref):
    x = x_ref[...]
    p = p_ref[...]
    h = jnp.dot(x, w0x_ref[...], preferred_element_type=jnp.float32)
    h += jnp.dot(p, w0p_ref[...], preferred_element_type=jnp.float32)
    h = jnp.maximum(h + b0_ref[...], 0.0)
    h = jnp.dot(h, w1_ref[...], preferred_element_type=jnp.float32)
    h = jnp.maximum(h + b1_ref[...], 0.0)
    h = jnp.dot(h, w2_ref[...], preferred_element_type=jnp.float32)
    h = h + b2_ref[...]
    h = h.reshape(n_cloud, q, h.shape[1])
    g = _max_mid(h)
    g = jnp.dot(g, h0_ref[...], preferred_element_type=jnp.float32)
    g = jnp.maximum(g + hb0_ref[...], 0.0)
    g = jnp.dot(g, h1_ref[...], preferred_element_type=jnp.float32)
    g = jnp.maximum(g + hb1_ref[...], 0.0)
    g = jnp.dot(g, h2_ref[...], preferred_element_type=jnp.float32)
    g = g + hb2_ref[...]
    mx = jnp.max(g, axis=1, keepdims=True)
    lse = jnp.log(jnp.sum(jnp.exp(g - mx), axis=1, keepdims=True)) + mx
    o_ref[...] = g - lse


def _sa3_head(x2d, p2d, n_cloud, q, args):
    ncls = args[-1].shape[1]
    return pl.pallas_call(
        functools.partial(_sa3_body, n_cloud, q),
        out_shape=jax.ShapeDtypeStruct((n_cloud, ncls), jnp.float32),
    )(x2d, p2d, *args)


def _fold_norm(params):
    s = (1.0 + _EPS) ** 0.5
    ws, bs = [], []
    n = len(params["W"])
    for i in range(n):
        w, b = params["W"][i], params["b"][i]
        if i < n - 1:
            g, beta = params["g"][i], params["beta"][i]
            ws.append(w * (g / s)[None, :])
            bs.append(b * g / s + beta)
        else:
            ws.append(w)
            bs.append(b)
    return ws, bs


def _select_neighbors(posq, px, py, pz, r):
    d2 = ((posq[:, :, 0:1] - px[:, None, :]) ** 2
          + (posq[:, :, 1:2] - py[:, None, :]) ** 2
          + (posq[:, :, 2:3] - pz[:, None, :]) ** 2)
    score = jnp.where(d2 <= r * r, -d2, -jnp.inf)
    vals, idx = jax.lax.top_k(score, _MAX_NB)
    maskf = jnp.where(vals > -jnp.inf, 0.0, -jnp.inf).astype(jnp.float32)
    return idx.astype(jnp.int32), maskf


def _sc_gather(u_flat, idx_flat, d):
    tot = idx_flat.shape[0]
    info = plsc.get_sparse_core_info()
    nw = info.num_cores * info.num_subcores
    rows_w = tot // nw
    ch = 512
    n_chunk = rows_w // ch
    mesh = plsc.VectorSubcoreMesh(core_axis_name="c", subcore_axis_name="s")

    @functools.partial(
        pl.kernel, mesh=mesh,
        out_type=jax.ShapeDtypeStruct((tot, d), jnp.float32),
        scratch_types=[
            pltpu.VMEM((ch,), jnp.int32),
            pltpu.VMEM((ch, d), jnp.float32),
            pltpu.SemaphoreType.DMA,
        ],
    )
    def gather_k(u_hbm, idx_hbm, out_hbm, idx_v, rows_v, sem):
        wid = lax.axis_index("s") * info.num_cores + lax.axis_index("c")
        base = wid * rows_w

        def chunk(ci, carry):
            off = base + ci * ch
            pltpu.sync_copy(idx_hbm.at[pl.ds(off, ch)], idx_v)
            pltpu.async_copy(u_hbm.at[idx_v], rows_v, sem).wait()
            pltpu.sync_copy(rows_v, out_hbm.at[pl.ds(off, ch)])
            return carry

        lax.fori_loop(0, n_chunk, chunk, 0)

    return gather_k(u_flat, idx_flat)


def _gather_pairs(u, idx):
    nb, q, k = idx.shape
    n, d = u.shape[1], u.shape[2]
    gbase = jnp.arange(nb, dtype=jnp.int32)[:, None, None] * n
    flat = (idx + gbase).reshape(nb * q * k)
    out = _sc_gather(u.reshape(nb * n, d), flat, d)
    return out.reshape(nb, q * k, d)


def _sa_stage(xb, posb, px, py, pz, params, ratio, r, qb):
    nb, n_pts, fdim = xb.shape
    n_samp = int(n_pts * ratio)
    ws, bs = _fold_norm(params)
    w1, w2, w3 = ws
    b1, b2, b3 = bs
    d1 = w1.shape[1]

    sel, qx, qy, qz = _fps(px, py, pz, n_samp)
    posq = jnp.stack([qx, qy, qz], axis=-1)

    u = _compute_u(xb.reshape(nb * n_pts, fdim),
                   posb.reshape(nb * n_pts, 3),
                   w1[:fdim], w1[fdim:]).reshape(nb, n_pts, d1)
    if d1 < 128:
        u = jnp.pad(u, ((0, 0), (0, 0), (0, 128 - d1)))

    idx, maskf = _select_neighbors(posq, px, py, pz, r)
    g = _gather_pairs(u, idx)

    x_out = _pair_mlp(g, posq, maskf, w1[fdim:], b1[None, :], w2,
                      b2[None, :], w3, b3[None, :], qb)
    return x_out, posq, qx, qy, qz


def kernel(x, pos, batch, params):
    nb = batch.shape[0] // 1024
    n_pts = 1024
    xb = x.reshape(nb, n_pts, -1)
    pb = pos.reshape(nb, n_pts, 3)
    px = pb[:, :, 0]
    py = pb[:, :, 1]
    pz = pb[:, :, 2]

    x1, posq1, q1x, q1y, q1z = _sa_stage(xb, pb, px, py, pz, params["sa1"],
                                         0.5, 0.2, 128)
    x2, posq2, _, _, _ = _sa_stage(x1, posq1, q1x, q1y, q1z, params["sa2"],
                                   0.25, 0.4, 128)

    ws, bs = _fold_norm(params["sa3"])
    hw, hb = _fold_norm(params["head"])
    q2 = x2.shape[1]
    f2 = x2.shape[2]
    args = (ws[0][:f2], ws[0][f2:], bs[0][None, :], ws[1], bs[1][None, :],
            ws[2], bs[2][None, :], hw[0], hb[0][None, :], hw[1],
            hb[1][None, :], hw[2], hb[2][None, :])
    return _sa3_head(x2.reshape(nb * q2, f2), posq2.reshape(nb * q2, 3),
                     nb, q2, args)
